# baseline (speedup 1.0000x reference)
import functools

import jax
import jax.numpy as jnp
from jax import lax
from jax.experimental import pallas as pl
from jax.experimental.pallas import tpu as pltpu
from jax.experimental.pallas import tpu_sc as plsc

N_NODES = 50000
D = 64
E = 800000

NPAD = 50176
HALF = NPAD // 2
ACC = HALF + 128
GARB = HALF
C = 128
NTILE = 16
G = 400
PT = G * C
EP = NTILE * PT
ER = EP // C
KI = 3
C2 = KI * C
G2 = G // KI
EBLK = 1536
SLOT0 = EP // 32
SLOTC = SLOT0 + EBLK + 128

_MESH = plsc.VectorSubcoreMesh(
    core_axis_name="c", subcore_axis_name="s", num_cores=2, num_subcores=16)
_SC_PARAMS = pltpu.CompilerParams(use_tc_tiling_on_sc=False)
_SC_PARAMS_NL = pltpu.CompilerParams(
    use_tc_tiling_on_sc=False, needs_layout_passes=False)


@functools.partial(
    pl.kernel,
    out_type=jax.ShapeDtypeStruct((NPAD, D), jnp.float32),
    mesh=_MESH,
    scratch_types=[
        pltpu.VMEM((C2,), jnp.int32),
        pltpu.VMEM((C2,), jnp.int32),
        pltpu.VMEM((C2,), jnp.int32),
        pltpu.VMEM((C2, D), jnp.float32),
        pltpu.VMEM((16,), jnp.int32),
        pltpu.VMEM_SHARED((ACC, D), jnp.float32),
        pltpu.SemaphoreType.DMA,
    ],
    compiler_params=_SC_PARAMS,
)
def _layer_sc(s_hbm, rowP_hbm, colP_hbm, cnt_hbm, z_hbm, out_hbm,
              rowb, colb, idxb, msg, cnt_v, acc_sh, sem):
    cid = lax.axis_index("c")
    sid = lax.axis_index("s")
    sc_lo = cid * HALF
    zrows = ACC // NTILE
    zr = sid * zrows
    pltpu.sync_copy(z_hbm.at[pl.ds(zr, zrows)], acc_sh.at[pl.ds(zr, zrows)])
    plsc.subcore_barrier()

    for k in range(2):
        w = 2 * sid + k
        pltpu.sync_copy(cnt_hbm.at[w], cnt_v)
        cv = cnt_v[...]
        cnt = jnp.where(cid == 0, cv[0], cv[1])
        nch = lax.div(cnt + (C2 - 1), C2)

        def chunk(g, carry):
            e0 = g * C2
            pltpu.sync_copy(rowP_hbm.at[cid, w, pl.ds(e0, C2)], rowb)
            pltpu.sync_copy(colP_hbm.at[cid, w, pl.ds(e0, C2)], colb)
            pltpu.async_copy(s_hbm.at[rowb], msg, sem).wait()
            for i in range(C2 // 16):
                v = colb[pl.ds(i * 16, 16)] - sc_lo
                oob = (v < 0) | (v >= HALF)
                idxb[pl.ds(i * 16, 16)] = jnp.where(oob, GARB, v)
            pltpu.sync_copy(msg, acc_sh.at[idxb], add=True)
            return carry

        lax.fori_loop(0, nch, chunk, 0)
    plsc.subcore_barrier()
    wrows = HALF // NTILE
    wr = sid * wrows
    pltpu.sync_copy(acc_sh.at[pl.ds(wr, wrows)],
                    out_hbm.at[pl.ds(sc_lo + wr, wrows)])


SLOT = EP // 32
PBLK = 1280
PNB = SLOT // PBLK
PNB_LAST = (E - 31 * SLOT) // PBLK
SLOTP = SLOTC


@functools.partial(
    pl.kernel,
    out_type=(jax.ShapeDtypeStruct((2, 32, SLOTC), jnp.int32),
              jax.ShapeDtypeStruct((2, 32, SLOTC), jnp.int32),
              jax.ShapeDtypeStruct((32, 16), jnp.int32)),
    mesh=_MESH,
    scratch_types=[
        pltpu.VMEM((PBLK,), jnp.int32),
        pltpu.VMEM((PBLK,), jnp.int32),
        pltpu.VMEM((SLOTP,), jnp.int32),
        pltpu.VMEM((SLOTP,), jnp.int32),
        pltpu.VMEM((SLOTP,), jnp.int32),
        pltpu.VMEM((SLOTP,), jnp.int32),
        pltpu.VMEM((16,), jnp.int32),
        pltpu.SMEM((2,), jnp.int32),
    ],
    compiler_params=_SC_PARAMS_NL,
)
def _part_sc(row_hbm, col_hbm, rowP, colP, cntO,
             rowI, colI, rowA, colA, rowB, colB, cnt_v, offs):
    cid = lax.axis_index("c")
    sid = lax.axis_index("s")
    w = cid * NTILE + sid
    ebase = w * SLOT

    zero16 = jnp.zeros((16,), jnp.int32)
    npad16 = jnp.full((16,), NPAD, jnp.int32)

    offs[0] = 0
    offs[1] = 0

    def block(b, carry):
        pltpu.sync_copy(row_hbm.at[pl.ds(ebase + b * PBLK, PBLK)], rowI)
        pltpu.sync_copy(col_hbm.at[pl.ds(ebase + b * PBLK, PBLK)], colI)
        for i in range(PBLK // 16):
            sl = pl.ds(i * 16, 16)
            rg = rowI[sl]
            cg = colI[sl]
            m = cg < HALF
            mi32 = jnp.where(m, 1, 0).astype(jnp.int32)
            n = jnp.sum(mi32)
            oA = offs[0]
            oB = offs[1]
            idxA = oA + plsc.cumsum(mi32) - 1
            plsc.store_scatter(rowA, [idxA], rg, mask=m)
            plsc.store_scatter(colA, [idxA], cg, mask=m)
            mb = jnp.logical_not(m)
            idxB = oB + plsc.cumsum(jnp.where(mb, 1, 0).astype(jnp.int32)) - 1
            plsc.store_scatter(rowB, [idxB], rg, mask=mb)
            plsc.store_scatter(colB, [idxB], cg, mask=mb)
            offs[0] = oA + n
            offs[1] = oB + (16 - n)
        return carry

    nb = jnp.where(w == 31, PNB_LAST, PNB)
    lax.fori_loop(0, nb, block, 0)
    oA = offs[0]
    oB = offs[1]
    iota16 = lax.iota(jnp.int32, 16)
    for t in range(EBLK // 16 + 1):
        plsc.store_scatter(colA, [oA + t * 16 + iota16], npad16)
        plsc.store_scatter(rowA, [oA + t * 16 + iota16], zero16)
        plsc.store_scatter(colB, [oB + t * 16 + iota16], npad16)
        plsc.store_scatter(rowB, [oB + t * 16 + iota16], zero16)
    pltpu.sync_copy(rowA.at[pl.ds(0, SLOTC)], rowP.at[0, w])
    pltpu.sync_copy(colA.at[pl.ds(0, SLOTC)], colP.at[0, w])
    pltpu.sync_copy(rowB.at[pl.ds(0, SLOTC)], rowP.at[1, w])
    pltpu.sync_copy(colB.at[pl.ds(0, SLOTC)], colP.at[1, w])
    cnt_v[...] = jnp.where(iota16 == 1, oB, oA).astype(jnp.int32)
    pltpu.sync_copy(cnt_v, cntO.at[w])


ACCD = NPAD + 128
DBLK = 20
DG = ER // 32
DNBLK = DG // DBLK


@functools.partial(
    pl.kernel,
    out_type=jax.ShapeDtypeStruct((2, NPAD, 16), jnp.float32),
    mesh=_MESH,
    scratch_types=[
        pltpu.VMEM((DBLK, C), jnp.int32),
        pltpu.VMEM((C, 16), jnp.float32),
        pltpu.VMEM_SHARED((ACCD, 16), jnp.float32),
        pltpu.SemaphoreType.DMA,
    ],
    compiler_params=_SC_PARAMS,
)
def _deg_sc(col_hbm, z16_hbm, ones_hbm, out_hbm, colb, ones_v, acc_sh, sem):
    cid = lax.axis_index("c")
    sid = lax.axis_index("s")
    zrows = ACCD // NTILE
    zr = sid * zrows
    pltpu.sync_copy(z16_hbm.at[pl.ds(zr, zrows)], acc_sh.at[pl.ds(zr, zrows)])
    pltpu.sync_copy(ones_hbm, ones_v)
    plsc.subcore_barrier()

    rbase = (cid * NTILE + sid) * DG

    def block(b, carry):
        pltpu.sync_copy(col_hbm.at[pl.ds(rbase + b * DBLK, DBLK)], colb)
        descs = [
            pltpu.async_copy(ones_v, acc_sh.at[colb.at[j]], sem, add=True)
            for j in range(DBLK)
        ]
        for d in descs:
            d.wait()
        return carry

    lax.fori_loop(0, DNBLK, block, 0)
    plsc.subcore_barrier()
    wrows = NPAD // NTILE
    wr = sid * wrows
    pltpu.sync_copy(acc_sh.at[pl.ds(wr, wrows)],
                    out_hbm.at[cid, pl.ds(wr, wrows)])


STRIPE = NPAD // 32
_CHUNKS = (320, 320, 320, 320, 288)


def _rsqrt16(d):
    di = plsc.bitcast(d, jnp.int32)
    x = plsc.bitcast(jnp.int32(0x5F3759DF) - (di >> 1), jnp.float32)
    for _ in range(3):
        x = x * (1.5 - 0.5 * d * x * x)
    return jnp.where(d > 0.0, x, 0.0)


def _row_scale(dv, buf, r):
    out = []
    for q in range(D // 16):
        out.append(dv * buf[r, pl.ds(q * 16, 16)])
    return out


@functools.partial(
    pl.kernel,
    out_type=(jax.ShapeDtypeStruct((NPAD, D), jnp.float32),
              jax.ShapeDtypeStruct((NPAD,), jnp.float32)),
    mesh=_MESH,
    scratch_types=[
        pltpu.VMEM((_CHUNKS[0], 16), jnp.float32),
        pltpu.VMEM((_CHUNKS[0], 16), jnp.float32),
        pltpu.VMEM((_CHUNKS[0], D), jnp.float32),
        pltpu.VMEM((_CHUNKS[0],), jnp.float32),
    ],
    compiler_params=_SC_PARAMS_NL,
)
def _norm_sc(deg_hbm, emb_hbm, s0_out, dis_out, p0b, p1b, embb, disb):
    cid = lax.axis_index("c")
    sid = lax.axis_index("s")
    wid = cid * NTILE + sid
    w31 = wid == 31
    r0 = wid * STRIPE
    iota16 = lax.iota(jnp.int32, 16)
    zero16 = jnp.zeros((16,), jnp.int32)
    off = 0
    for rows in _CHUNKS:
        base = r0 + off
        pltpu.sync_copy(deg_hbm.at[0, pl.ds(base, rows)], p0b.at[pl.ds(0, rows)])
        pltpu.sync_copy(deg_hbm.at[1, pl.ds(base, rows)], p1b.at[pl.ds(0, rows)])
        if off + rows == STRIPE:
            last = N_NODES - 31 * STRIPE - off

            @pl.when(w31)
            def _():
                pltpu.sync_copy(emb_hbm.at[pl.ds(base, last)],
                                embb.at[pl.ds(0, last)])

            @pl.when(jnp.logical_not(w31))
            def _():
                pltpu.sync_copy(emb_hbm.at[pl.ds(base, rows)],
                                embb.at[pl.ds(0, rows)])
        else:
            pltpu.sync_copy(emb_hbm.at[pl.ds(base, rows)],
                            embb.at[pl.ds(0, rows)])

        def grp(g, carry):
            d16 = (plsc.load_gather(p0b, [g * 16 + iota16, zero16])
                   + plsc.load_gather(p1b, [g * 16 + iota16, zero16]))
            disb[pl.ds(g * 16, 16)] = _rsqrt16(d16)
            return carry

        lax.fori_loop(0, rows // 16, grp, 0)

        def rowgrp(g, carry):
            dv16 = disb[pl.ds(g * 16, 16)]
            for l in range(16):
                r = g * 16 + l
                dv = dv16[l]
                for q in range(D // 16):
                    embb[r, pl.ds(q * 16, 16)] = dv * embb[r, pl.ds(q * 16, 16)]
            return carry

        lax.fori_loop(0, rows // 16, rowgrp, 0)
        pltpu.sync_copy(embb.at[pl.ds(0, rows)], s0_out.at[pl.ds(base, rows)])
        pltpu.sync_copy(disb.at[pl.ds(0, rows)], dis_out.at[pl.ds(base, rows)])
        off += rows


def _make_scale(final):
    outs = (jax.ShapeDtypeStruct((N_NODES, D), jnp.float32)
            if final else
            (jax.ShapeDtypeStruct((NPAD, D), jnp.float32),
             jax.ShapeDtypeStruct((NPAD, D), jnp.float32)))

    @functools.partial(
        pl.kernel,
        out_type=outs,
        mesh=_MESH,
        scratch_types=[
            pltpu.VMEM((_CHUNKS[0], D), jnp.float32),
            pltpu.VMEM((_CHUNKS[0], D), jnp.float32),
            pltpu.VMEM((_CHUNKS[0],), jnp.float32),
        ],
        compiler_params=_SC_PARAMS_NL,
    )
    def _scale(acc_hbm, dis_hbm, sum_hbm, *refs):
        if final:
            out_hbm, accb, sumb, disb = refs
            s_out = sum_out = None
        else:
            s_out, sum_out, accb, sumb, disb = refs
        cid = lax.axis_index("c")
        sid = lax.axis_index("s")
        wid = cid * NTILE + sid
        w31 = wid == 31
        raw_sum = sum_hbm.shape[0] == N_NODES
        r0 = wid * STRIPE
        off = 0
        for rows in _CHUNKS:
            base = r0 + off
            pltpu.sync_copy(acc_hbm.at[pl.ds(base, rows)], accb.at[pl.ds(0, rows)])
            if raw_sum and off + rows == STRIPE:
                last = N_NODES - 31 * STRIPE - off

                @pl.when(w31)
                def _():
                    pltpu.sync_copy(sum_hbm.at[pl.ds(base, last)],
                                    sumb.at[pl.ds(0, last)])

                @pl.when(jnp.logical_not(w31))
                def _():
                    pltpu.sync_copy(sum_hbm.at[pl.ds(base, rows)],
                                    sumb.at[pl.ds(0, rows)])
            else:
                pltpu.sync_copy(sum_hbm.at[pl.ds(base, rows)],
                                sumb.at[pl.ds(0, rows)])
            pltpu.sync_copy(dis_hbm.at[pl.ds(base, rows)], disb.at[pl.ds(0, rows)])

            def rowgrp(g, carry):
                dv16 = disb[pl.ds(g * 16, 16)]
                for l in range(16):
                    r = g * 16 + l
                    dv = dv16[l]
                    for q in range(D // 16):
                        sl = pl.ds(q * 16, 16)
                        da = dv * accb[r, sl]
                        if final:
                            sumb[r, sl] = (sumb[r, sl] + da) * 0.25
                        else:
                            sumb[r, sl] = sumb[r, sl] + da
                            accb[r, sl] = dv * da
                return carry

            lax.fori_loop(0, rows // 16, rowgrp, 0)
            if final:
                if off + rows == STRIPE:
                    last = N_NODES - 31 * STRIPE - off

                    @pl.when(w31)
                    def _():
                        pltpu.sync_copy(sumb.at[pl.ds(0, last)],
                                        out_hbm.at[pl.ds(base, last)])

                    @pl.when(jnp.logical_not(w31))
                    def _():
                        pltpu.sync_copy(sumb.at[pl.ds(0, rows)],
                                        out_hbm.at[pl.ds(base, rows)])
                else:
                    pltpu.sync_copy(sumb.at[pl.ds(0, rows)],
                                    out_hbm.at[pl.ds(base, rows)])
            else:
                pltpu.sync_copy(accb.at[pl.ds(0, rows)],
                                s_out.at[pl.ds(base, rows)])
                pltpu.sync_copy(sumb.at[pl.ds(0, rows)],
                                sum_out.at[pl.ds(base, rows)])
            off += rows

    return _scale


_scale_sc = _make_scale(False)
_final_sc = _make_scale(True)


def kernel(edge_index, embedding):
    row = edge_index[0].astype(jnp.int32)
    col = edge_index[1].astype(jnp.int32)
    pad_e = EP - E
    col_p2 = jnp.concatenate(
        [col, jnp.full((pad_e,), NPAD, jnp.int32)]).reshape(ER, C)
    z64 = jnp.zeros((ACC, D), jnp.float32)
    z16 = jnp.zeros((ACCD, 16), jnp.float32)
    ones16 = jnp.ones((C, 16), jnp.float32)

    degt = _deg_sc(col_p2, z16, ones16)
    rowP, colP, cntP = _part_sc(row, col)
    s0, dis = _norm_sc(degt, embedding)
    acc1 = _layer_sc(s0, rowP, colP, cntP, z64)
    s1, summ = _scale_sc(acc1, dis, embedding)
    acc2 = _layer_sc(s1, rowP, colP, cntP, z64)
    s2, summ = _scale_sc(acc2, dis, summ)
    acc3 = _layer_sc(s2, rowP, colP, cntP, z64)
    return _final_sc(acc3, dis, summ)


# speedup vs baseline: 1.0143x; 1.0143x over previous
import functools

import jax
import jax.numpy as jnp
from jax import lax
from jax.experimental import pallas as pl
from jax.experimental.pallas import tpu as pltpu
from jax.experimental.pallas import tpu_sc as plsc

N_NODES = 50000
D = 64
E = 800000

NPAD = 50176
HALF = NPAD // 2
ACC = HALF + 128
GARB = HALF
C = 128
NTILE = 16
G = 400
PT = G * C
EP = NTILE * PT
ER = EP // C
KI = 1
C2 = KI * C
G2 = G // KI
EBLK = 1536
SLOT0 = EP // 32
SLOTC = SLOT0 + EBLK + 128

_MESH = plsc.VectorSubcoreMesh(
    core_axis_name="c", subcore_axis_name="s", num_cores=2, num_subcores=16)
_SC_PARAMS = pltpu.CompilerParams(use_tc_tiling_on_sc=False)
_SC_PARAMS_NL = pltpu.CompilerParams(
    use_tc_tiling_on_sc=False, needs_layout_passes=False)


@functools.partial(
    pl.kernel,
    out_type=jax.ShapeDtypeStruct((NPAD, D), jnp.float32),
    mesh=_MESH,
    scratch_types=[
        pltpu.VMEM((C2,), jnp.int32),
        pltpu.VMEM((C2,), jnp.int32),
        pltpu.VMEM((C2,), jnp.int32),
        pltpu.VMEM((C2, D), jnp.float32),
        pltpu.VMEM((16,), jnp.int32),
        pltpu.VMEM_SHARED((ACC, D), jnp.float32),
        pltpu.SemaphoreType.DMA,
    ],
    compiler_params=_SC_PARAMS,
)
def _layer_sc(s_hbm, rowP_hbm, colP_hbm, cnt_hbm, z_hbm, out_hbm,
              rowb, colb, idxb, msg, cnt_v, acc_sh, sem):
    cid = lax.axis_index("c")
    sid = lax.axis_index("s")
    sc_lo = cid * HALF
    zrows = ACC // NTILE
    zr = sid * zrows
    pltpu.sync_copy(z_hbm.at[pl.ds(zr, zrows)], acc_sh.at[pl.ds(zr, zrows)])
    plsc.subcore_barrier()

    for k in range(2):
        w = 2 * sid + k
        pltpu.sync_copy(cnt_hbm.at[w], cnt_v)
        cv = cnt_v[...]
        cnt = jnp.where(cid == 0, cv[0], cv[1])
        nch = lax.div(cnt + (C2 - 1), C2)

        def chunk(g, carry):
            e0 = g * C2
            pltpu.sync_copy(rowP_hbm.at[cid, w, pl.ds(e0, C2)], rowb)
            pltpu.sync_copy(colP_hbm.at[cid, w, pl.ds(e0, C2)], colb)
            pltpu.async_copy(s_hbm.at[rowb], msg, sem).wait()
            for i in range(C2 // 16):
                v = colb[pl.ds(i * 16, 16)] - sc_lo
                oob = (v < 0) | (v >= HALF)
                idxb[pl.ds(i * 16, 16)] = jnp.where(oob, GARB, v)
            pltpu.sync_copy(msg, acc_sh.at[idxb], add=True)
            return carry

        lax.fori_loop(0, nch, chunk, 0)
    plsc.subcore_barrier()
    wrows = HALF // NTILE
    wr = sid * wrows
    pltpu.sync_copy(acc_sh.at[pl.ds(wr, wrows)],
                    out_hbm.at[pl.ds(sc_lo + wr, wrows)])


SLOT = EP // 32
PBLK = 1280
PNB = SLOT // PBLK
PNB_LAST = (E - 31 * SLOT) // PBLK
SLOTP = SLOTC


@functools.partial(
    pl.kernel,
    out_type=(jax.ShapeDtypeStruct((2, 32, SLOTC), jnp.int32),
              jax.ShapeDtypeStruct((2, 32, SLOTC), jnp.int32),
              jax.ShapeDtypeStruct((32, 16), jnp.int32)),
    mesh=_MESH,
    scratch_types=[
        pltpu.VMEM((PBLK,), jnp.int32),
        pltpu.VMEM((PBLK,), jnp.int32),
        pltpu.VMEM((SLOTP,), jnp.int32),
        pltpu.VMEM((SLOTP,), jnp.int32),
        pltpu.VMEM((SLOTP,), jnp.int32),
        pltpu.VMEM((SLOTP,), jnp.int32),
        pltpu.VMEM((16,), jnp.int32),
        pltpu.SMEM((2,), jnp.int32),
    ],
    compiler_params=_SC_PARAMS_NL,
)
def _part_sc(row_hbm, col_hbm, rowP, colP, cntO,
             rowI, colI, rowA, colA, rowB, colB, cnt_v, offs):
    cid = lax.axis_index("c")
    sid = lax.axis_index("s")
    w = cid * NTILE + sid
    ebase = w * SLOT

    zero16 = jnp.zeros((16,), jnp.int32)
    npad16 = jnp.full((16,), NPAD, jnp.int32)

    offs[0] = 0
    offs[1] = 0

    def block(b, carry):
        pltpu.sync_copy(row_hbm.at[pl.ds(ebase + b * PBLK, PBLK)], rowI)
        pltpu.sync_copy(col_hbm.at[pl.ds(ebase + b * PBLK, PBLK)], colI)
        for i in range(PBLK // 16):
            sl = pl.ds(i * 16, 16)
            rg = rowI[sl]
            cg = colI[sl]
            m = cg < HALF
            mi32 = jnp.where(m, 1, 0).astype(jnp.int32)
            n = jnp.sum(mi32)
            oA = offs[0]
            oB = offs[1]
            idxA = oA + plsc.cumsum(mi32) - 1
            plsc.store_scatter(rowA, [idxA], rg, mask=m)
            plsc.store_scatter(colA, [idxA], cg, mask=m)
            mb = jnp.logical_not(m)
            idxB = oB + plsc.cumsum(jnp.where(mb, 1, 0).astype(jnp.int32)) - 1
            plsc.store_scatter(rowB, [idxB], rg, mask=mb)
            plsc.store_scatter(colB, [idxB], cg, mask=mb)
            offs[0] = oA + n
            offs[1] = oB + (16 - n)
        return carry

    nb = jnp.where(w == 31, PNB_LAST, PNB)
    lax.fori_loop(0, nb, block, 0)
    oA = offs[0]
    oB = offs[1]
    iota16 = lax.iota(jnp.int32, 16)
    for t in range(EBLK // 16 + 1):
        plsc.store_scatter(colA, [oA + t * 16 + iota16], npad16)
        plsc.store_scatter(rowA, [oA + t * 16 + iota16], zero16)
        plsc.store_scatter(colB, [oB + t * 16 + iota16], npad16)
        plsc.store_scatter(rowB, [oB + t * 16 + iota16], zero16)
    pltpu.sync_copy(rowA.at[pl.ds(0, SLOTC)], rowP.at[0, w])
    pltpu.sync_copy(colA.at[pl.ds(0, SLOTC)], colP.at[0, w])
    pltpu.sync_copy(rowB.at[pl.ds(0, SLOTC)], rowP.at[1, w])
    pltpu.sync_copy(colB.at[pl.ds(0, SLOTC)], colP.at[1, w])
    cnt_v[...] = jnp.where(iota16 == 1, oB, oA).astype(jnp.int32)
    pltpu.sync_copy(cnt_v, cntO.at[w])


ACCD = NPAD + 128
DBLK = 20
DG = ER // 32
DNBLK = DG // DBLK


@functools.partial(
    pl.kernel,
    out_type=jax.ShapeDtypeStruct((2, NPAD, 16), jnp.float32),
    mesh=_MESH,
    scratch_types=[
        pltpu.VMEM((DBLK, C), jnp.int32),
        pltpu.VMEM((C, 16), jnp.float32),
        pltpu.VMEM_SHARED((ACCD, 16), jnp.float32),
        pltpu.SemaphoreType.DMA,
    ],
    compiler_params=_SC_PARAMS,
)
def _deg_sc(col_hbm, z16_hbm, ones_hbm, out_hbm, colb, ones_v, acc_sh, sem):
    cid = lax.axis_index("c")
    sid = lax.axis_index("s")
    zrows = ACCD // NTILE
    zr = sid * zrows
    pltpu.sync_copy(z16_hbm.at[pl.ds(zr, zrows)], acc_sh.at[pl.ds(zr, zrows)])
    pltpu.sync_copy(ones_hbm, ones_v)
    plsc.subcore_barrier()

    rbase = (cid * NTILE + sid) * DG

    def block(b, carry):
        pltpu.sync_copy(col_hbm.at[pl.ds(rbase + b * DBLK, DBLK)], colb)
        descs = [
            pltpu.async_copy(ones_v, acc_sh.at[colb.at[j]], sem, add=True)
            for j in range(DBLK)
        ]
        for d in descs:
            d.wait()
        return carry

    lax.fori_loop(0, DNBLK, block, 0)
    plsc.subcore_barrier()
    wrows = NPAD // NTILE
    wr = sid * wrows
    pltpu.sync_copy(acc_sh.at[pl.ds(wr, wrows)],
                    out_hbm.at[cid, pl.ds(wr, wrows)])


STRIPE = NPAD // 32
_CHUNKS = (320, 320, 320, 320, 288)


def _rsqrt16(d):
    di = plsc.bitcast(d, jnp.int32)
    x = plsc.bitcast(jnp.int32(0x5F3759DF) - (di >> 1), jnp.float32)
    for _ in range(3):
        x = x * (1.5 - 0.5 * d * x * x)
    return jnp.where(d > 0.0, x, 0.0)


def _row_scale(dv, buf, r):
    out = []
    for q in range(D // 16):
        out.append(dv * buf[r, pl.ds(q * 16, 16)])
    return out


@functools.partial(
    pl.kernel,
    out_type=(jax.ShapeDtypeStruct((NPAD, D), jnp.float32),
              jax.ShapeDtypeStruct((NPAD,), jnp.float32)),
    mesh=_MESH,
    scratch_types=[
        pltpu.VMEM((_CHUNKS[0], 16), jnp.float32),
        pltpu.VMEM((_CHUNKS[0], 16), jnp.float32),
        pltpu.VMEM((_CHUNKS[0], D), jnp.float32),
        pltpu.VMEM((_CHUNKS[0],), jnp.float32),
    ],
    compiler_params=_SC_PARAMS_NL,
)
def _norm_sc(deg_hbm, emb_hbm, s0_out, dis_out, p0b, p1b, embb, disb):
    cid = lax.axis_index("c")
    sid = lax.axis_index("s")
    wid = cid * NTILE + sid
    w31 = wid == 31
    r0 = wid * STRIPE
    iota16 = lax.iota(jnp.int32, 16)
    zero16 = jnp.zeros((16,), jnp.int32)
    off = 0
    for rows in _CHUNKS:
        base = r0 + off
        pltpu.sync_copy(deg_hbm.at[0, pl.ds(base, rows)], p0b.at[pl.ds(0, rows)])
        pltpu.sync_copy(deg_hbm.at[1, pl.ds(base, rows)], p1b.at[pl.ds(0, rows)])
        if off + rows == STRIPE:
            last = N_NODES - 31 * STRIPE - off

            @pl.when(w31)
            def _():
                pltpu.sync_copy(emb_hbm.at[pl.ds(base, last)],
                                embb.at[pl.ds(0, last)])

            @pl.when(jnp.logical_not(w31))
            def _():
                pltpu.sync_copy(emb_hbm.at[pl.ds(base, rows)],
                                embb.at[pl.ds(0, rows)])
        else:
            pltpu.sync_copy(emb_hbm.at[pl.ds(base, rows)],
                            embb.at[pl.ds(0, rows)])

        def grp(g, carry):
            d16 = (plsc.load_gather(p0b, [g * 16 + iota16, zero16])
                   + plsc.load_gather(p1b, [g * 16 + iota16, zero16]))
            disb[pl.ds(g * 16, 16)] = _rsqrt16(d16)
            return carry

        lax.fori_loop(0, rows // 16, grp, 0)

        def rowgrp(g, carry):
            dv16 = disb[pl.ds(g * 16, 16)]
            for l in range(16):
                r = g * 16 + l
                dv = dv16[l]
                for q in range(D // 16):
                    embb[r, pl.ds(q * 16, 16)] = dv * embb[r, pl.ds(q * 16, 16)]
            return carry

        lax.fori_loop(0, rows // 16, rowgrp, 0)
        pltpu.sync_copy(embb.at[pl.ds(0, rows)], s0_out.at[pl.ds(base, rows)])
        pltpu.sync_copy(disb.at[pl.ds(0, rows)], dis_out.at[pl.ds(base, rows)])
        off += rows


def _make_scale(final):
    outs = (jax.ShapeDtypeStruct((N_NODES, D), jnp.float32)
            if final else
            (jax.ShapeDtypeStruct((NPAD, D), jnp.float32),
             jax.ShapeDtypeStruct((NPAD, D), jnp.float32)))

    @functools.partial(
        pl.kernel,
        out_type=outs,
        mesh=_MESH,
        scratch_types=[
            pltpu.VMEM((_CHUNKS[0], D), jnp.float32),
            pltpu.VMEM((_CHUNKS[0], D), jnp.float32),
            pltpu.VMEM((_CHUNKS[0],), jnp.float32),
        ],
        compiler_params=_SC_PARAMS_NL,
    )
    def _scale(acc_hbm, dis_hbm, sum_hbm, *refs):
        if final:
            out_hbm, accb, sumb, disb = refs
            s_out = sum_out = None
        else:
            s_out, sum_out, accb, sumb, disb = refs
        cid = lax.axis_index("c")
        sid = lax.axis_index("s")
        wid = cid * NTILE + sid
        w31 = wid == 31
        raw_sum = sum_hbm.shape[0] == N_NODES
        r0 = wid * STRIPE
        off = 0
        for rows in _CHUNKS:
            base = r0 + off
            pltpu.sync_copy(acc_hbm.at[pl.ds(base, rows)], accb.at[pl.ds(0, rows)])
            if raw_sum and off + rows == STRIPE:
                last = N_NODES - 31 * STRIPE - off

                @pl.when(w31)
                def _():
                    pltpu.sync_copy(sum_hbm.at[pl.ds(base, last)],
                                    sumb.at[pl.ds(0, last)])

                @pl.when(jnp.logical_not(w31))
                def _():
                    pltpu.sync_copy(sum_hbm.at[pl.ds(base, rows)],
                                    sumb.at[pl.ds(0, rows)])
            else:
                pltpu.sync_copy(sum_hbm.at[pl.ds(base, rows)],
                                sumb.at[pl.ds(0, rows)])
            pltpu.sync_copy(dis_hbm.at[pl.ds(base, rows)], disb.at[pl.ds(0, rows)])

            def rowgrp(g, carry):
                dv16 = disb[pl.ds(g * 16, 16)]
                for l in range(16):
                    r = g * 16 + l
                    dv = dv16[l]
                    for q in range(D // 16):
                        sl = pl.ds(q * 16, 16)
                        da = dv * accb[r, sl]
                        if final:
                            sumb[r, sl] = (sumb[r, sl] + da) * 0.25
                        else:
                            sumb[r, sl] = sumb[r, sl] + da
                            accb[r, sl] = dv * da
                return carry

            lax.fori_loop(0, rows // 16, rowgrp, 0)
            if final:
                if off + rows == STRIPE:
                    last = N_NODES - 31 * STRIPE - off

                    @pl.when(w31)
                    def _():
                        pltpu.sync_copy(sumb.at[pl.ds(0, last)],
                                        out_hbm.at[pl.ds(base, last)])

                    @pl.when(jnp.logical_not(w31))
                    def _():
                        pltpu.sync_copy(sumb.at[pl.ds(0, rows)],
                                        out_hbm.at[pl.ds(base, rows)])
                else:
                    pltpu.sync_copy(sumb.at[pl.ds(0, rows)],
                                    out_hbm.at[pl.ds(base, rows)])
            else:
                pltpu.sync_copy(accb.at[pl.ds(0, rows)],
                                s_out.at[pl.ds(base, rows)])
                pltpu.sync_copy(sumb.at[pl.ds(0, rows)],
                                sum_out.at[pl.ds(base, rows)])
            off += rows

    return _scale


_scale_sc = _make_scale(False)
_final_sc = _make_scale(True)


def kernel(edge_index, embedding):
    row = edge_index[0].astype(jnp.int32)
    col = edge_index[1].astype(jnp.int32)
    pad_e = EP - E
    col_p2 = jnp.concatenate(
        [col, jnp.full((pad_e,), NPAD, jnp.int32)]).reshape(ER, C)
    z64 = jnp.zeros((ACC, D), jnp.float32)
    z16 = jnp.zeros((ACCD, 16), jnp.float32)
    ones16 = jnp.ones((C, 16), jnp.float32)

    degt = _deg_sc(col_p2, z16, ones16)
    rowP, colP, cntP = _part_sc(row, col)
    s0, dis = _norm_sc(degt, embedding)
    acc1 = _layer_sc(s0, rowP, colP, cntP, z64)
    s1, summ = _scale_sc(acc1, dis, embedding)
    acc2 = _layer_sc(s1, rowP, colP, cntP, z64)
    s2, summ = _scale_sc(acc2, dis, summ)
    acc3 = _layer_sc(s2, rowP, colP, cntP, z64)
    return _final_sc(acc3, dis, summ)


# speedup vs baseline: 1.1413x; 1.1252x over previous
import functools

import jax
import jax.numpy as jnp
from jax import lax
from jax.experimental import pallas as pl
from jax.experimental.pallas import tpu as pltpu
from jax.experimental.pallas import tpu_sc as plsc

N_NODES = 50000
D = 64
E = 800000

NPAD = 50176
HALF = NPAD // 2
ACC = HALF + 128
GARB = HALF
C = 128
NTILE = 16
G = 400
PT = G * C
EP = NTILE * PT
ER = EP // C
KI = 2
C2 = KI * C
G2 = G // KI
EBLK = 1536
SLOT0 = EP // 32
SLOTC = SLOT0 + EBLK + 128

_MESH = plsc.VectorSubcoreMesh(
    core_axis_name="c", subcore_axis_name="s", num_cores=2, num_subcores=16)
_SC_PARAMS = pltpu.CompilerParams(use_tc_tiling_on_sc=False)
_SC_PARAMS_NL = pltpu.CompilerParams(
    use_tc_tiling_on_sc=False, needs_layout_passes=False)


@functools.partial(
    pl.kernel,
    out_type=jax.ShapeDtypeStruct((NPAD, D), jnp.float32),
    mesh=_MESH,
    scratch_types=[
        pltpu.VMEM((C2,), jnp.int32),
        pltpu.VMEM((C2,), jnp.int32),
        pltpu.VMEM((C2,), jnp.int32),
        pltpu.VMEM((C2, D), jnp.float32),
        pltpu.VMEM((16,), jnp.int32),
        pltpu.VMEM_SHARED((ACC, D), jnp.float32),
        pltpu.SemaphoreType.DMA,
    ],
    compiler_params=_SC_PARAMS,
)
def _layer_sc(s_hbm, rowP_hbm, colP_hbm, cnt_hbm, z_hbm, out_hbm,
              rowb, colb, idxb, msg, cnt_v, acc_sh, sem):
    cid = lax.axis_index("c")
    sid = lax.axis_index("s")
    sc_lo = cid * HALF
    zrows = ACC // NTILE
    zr = sid * zrows
    pltpu.sync_copy(z_hbm.at[pl.ds(zr, zrows)], acc_sh.at[pl.ds(zr, zrows)])
    plsc.subcore_barrier()

    for k in range(2):
        w = 2 * sid + k
        pltpu.sync_copy(cnt_hbm.at[w], cnt_v)
        cv = cnt_v[...]
        cnt = jnp.where(cid == 0, cv[0], cv[1])
        nch = lax.div(cnt + (C2 - 1), C2)

        def chunk(g, carry):
            e0 = g * C2
            pltpu.sync_copy(rowP_hbm.at[cid, w, pl.ds(e0, C2)], rowb)
            pltpu.sync_copy(colP_hbm.at[cid, w, pl.ds(e0, C2)], colb)
            pltpu.async_copy(s_hbm.at[rowb], msg, sem).wait()
            for i in range(C2 // 16):
                v = colb[pl.ds(i * 16, 16)] - sc_lo
                oob = (v < 0) | (v >= HALF)
                idxb[pl.ds(i * 16, 16)] = jnp.where(oob, GARB, v)
            pltpu.sync_copy(msg, acc_sh.at[idxb], add=True)
            return carry

        lax.fori_loop(0, nch, chunk, 0)
    plsc.subcore_barrier()
    wrows = HALF // NTILE
    wr = sid * wrows
    pltpu.sync_copy(acc_sh.at[pl.ds(wr, wrows)],
                    out_hbm.at[pl.ds(sc_lo + wr, wrows)])


SLOT = EP // 32
PBLK = 1280
PNB = SLOT // PBLK
PNB_LAST = (E - 31 * SLOT) // PBLK
SLOTP = SLOTC


@functools.partial(
    pl.kernel,
    out_type=(jax.ShapeDtypeStruct((2, 32, SLOTC), jnp.int32),
              jax.ShapeDtypeStruct((2, 32, SLOTC), jnp.int32),
              jax.ShapeDtypeStruct((32, 16), jnp.int32)),
    mesh=_MESH,
    scratch_types=[
        pltpu.VMEM((PBLK,), jnp.int32),
        pltpu.VMEM((PBLK,), jnp.int32),
        pltpu.VMEM((SLOTP,), jnp.int32),
        pltpu.VMEM((SLOTP,), jnp.int32),
        pltpu.VMEM((SLOTP,), jnp.int32),
        pltpu.VMEM((SLOTP,), jnp.int32),
        pltpu.VMEM((16,), jnp.int32),
        pltpu.SMEM((2,), jnp.int32),
    ],
    compiler_params=_SC_PARAMS_NL,
)
def _part_sc(row_hbm, col_hbm, rowP, colP, cntO,
             rowI, colI, rowA, colA, rowB, colB, cnt_v, offs):
    cid = lax.axis_index("c")
    sid = lax.axis_index("s")
    w = cid * NTILE + sid
    ebase = w * SLOT

    zero16 = jnp.zeros((16,), jnp.int32)
    npad16 = jnp.full((16,), NPAD, jnp.int32)

    offs[0] = 0
    offs[1] = 0

    def block(b, carry):
        pltpu.sync_copy(row_hbm.at[pl.ds(ebase + b * PBLK, PBLK)], rowI)
        pltpu.sync_copy(col_hbm.at[pl.ds(ebase + b * PBLK, PBLK)], colI)
        for i in range(PBLK // 16):
            sl = pl.ds(i * 16, 16)
            rg = rowI[sl]
            cg = colI[sl]
            m = cg < HALF
            mi32 = jnp.where(m, 1, 0).astype(jnp.int32)
            n = jnp.sum(mi32)
            oA = offs[0]
            oB = offs[1]
            idxA = oA + plsc.cumsum(mi32) - 1
            plsc.store_scatter(rowA, [idxA], rg, mask=m)
            plsc.store_scatter(colA, [idxA], cg, mask=m)
            mb = jnp.logical_not(m)
            idxB = oB + plsc.cumsum(jnp.where(mb, 1, 0).astype(jnp.int32)) - 1
            plsc.store_scatter(rowB, [idxB], rg, mask=mb)
            plsc.store_scatter(colB, [idxB], cg, mask=mb)
            offs[0] = oA + n
            offs[1] = oB + (16 - n)
        return carry

    nb = jnp.where(w == 31, PNB_LAST, PNB)
    lax.fori_loop(0, nb, block, 0)
    oA = offs[0]
    oB = offs[1]
    iota16 = lax.iota(jnp.int32, 16)
    for t in range(EBLK // 16 + 1):
        plsc.store_scatter(colA, [oA + t * 16 + iota16], npad16)
        plsc.store_scatter(rowA, [oA + t * 16 + iota16], zero16)
        plsc.store_scatter(colB, [oB + t * 16 + iota16], npad16)
        plsc.store_scatter(rowB, [oB + t * 16 + iota16], zero16)
    pltpu.sync_copy(rowA.at[pl.ds(0, SLOTC)], rowP.at[0, w])
    pltpu.sync_copy(colA.at[pl.ds(0, SLOTC)], colP.at[0, w])
    pltpu.sync_copy(rowB.at[pl.ds(0, SLOTC)], rowP.at[1, w])
    pltpu.sync_copy(colB.at[pl.ds(0, SLOTC)], colP.at[1, w])
    cnt_v[...] = jnp.where(iota16 == 1, oB, oA).astype(jnp.int32)
    pltpu.sync_copy(cnt_v, cntO.at[w])


ACCD = NPAD + 128
DBLK = 20
DG = ER // 32
DNBLK = DG // DBLK


@functools.partial(
    pl.kernel,
    out_type=jax.ShapeDtypeStruct((2, NPAD, 16), jnp.float32),
    mesh=_MESH,
    scratch_types=[
        pltpu.VMEM((DBLK, C), jnp.int32),
        pltpu.VMEM((C, 16), jnp.float32),
        pltpu.VMEM_SHARED((ACCD, 16), jnp.float32),
        pltpu.SemaphoreType.DMA,
    ],
    compiler_params=_SC_PARAMS,
)
def _deg_sc(col_hbm, z16_hbm, ones_hbm, out_hbm, colb, ones_v, acc_sh, sem):
    cid = lax.axis_index("c")
    sid = lax.axis_index("s")
    zrows = ACCD // NTILE
    zr = sid * zrows
    pltpu.sync_copy(z16_hbm.at[pl.ds(zr, zrows)], acc_sh.at[pl.ds(zr, zrows)])
    pltpu.sync_copy(ones_hbm, ones_v)
    plsc.subcore_barrier()

    rbase = (cid * NTILE + sid) * DG

    def block(b, carry):
        pltpu.sync_copy(col_hbm.at[pl.ds(rbase + b * DBLK, DBLK)], colb)
        descs = [
            pltpu.async_copy(ones_v, acc_sh.at[colb.at[j]], sem, add=True)
            for j in range(DBLK)
        ]
        for d in descs:
            d.wait()
        return carry

    lax.fori_loop(0, DNBLK, block, 0)
    plsc.subcore_barrier()
    wrows = NPAD // NTILE
    wr = sid * wrows
    pltpu.sync_copy(acc_sh.at[pl.ds(wr, wrows)],
                    out_hbm.at[cid, pl.ds(wr, wrows)])


STRIPE = NPAD // 32
_CHUNKS = (320, 320, 320, 320, 288)


def _rsqrt16(d):
    di = plsc.bitcast(d, jnp.int32)
    x = plsc.bitcast(jnp.int32(0x5F3759DF) - (di >> 1), jnp.float32)
    for _ in range(3):
        x = x * (1.5 - 0.5 * d * x * x)
    return jnp.where(d > 0.0, x, 0.0)


def _row_scale(dv, buf, r):
    out = []
    for q in range(D // 16):
        out.append(dv * buf[r, pl.ds(q * 16, 16)])
    return out


@functools.partial(
    pl.kernel,
    out_type=(jax.ShapeDtypeStruct((NPAD, D), jnp.float32),
              jax.ShapeDtypeStruct((NPAD,), jnp.float32)),
    mesh=_MESH,
    scratch_types=[
        pltpu.VMEM((_CHUNKS[0], 16), jnp.float32),
        pltpu.VMEM((_CHUNKS[0], 16), jnp.float32),
        pltpu.VMEM((_CHUNKS[0], D), jnp.float32),
        pltpu.VMEM((_CHUNKS[0],), jnp.float32),
    ],
    compiler_params=_SC_PARAMS_NL,
)
def _norm_sc(deg_hbm, emb_hbm, s0_out, dis_out, p0b, p1b, embb, disb):
    cid = lax.axis_index("c")
    sid = lax.axis_index("s")
    wid = cid * NTILE + sid
    w31 = wid == 31
    r0 = wid * STRIPE
    iota16 = lax.iota(jnp.int32, 16)
    zero16 = jnp.zeros((16,), jnp.int32)
    off = 0
    for rows in _CHUNKS:
        base = r0 + off
        pltpu.sync_copy(deg_hbm.at[0, pl.ds(base, rows)], p0b.at[pl.ds(0, rows)])
        pltpu.sync_copy(deg_hbm.at[1, pl.ds(base, rows)], p1b.at[pl.ds(0, rows)])
        if off + rows == STRIPE:
            last = N_NODES - 31 * STRIPE - off

            @pl.when(w31)
            def _():
                pltpu.sync_copy(emb_hbm.at[pl.ds(base, last)],
                                embb.at[pl.ds(0, last)])

            @pl.when(jnp.logical_not(w31))
            def _():
                pltpu.sync_copy(emb_hbm.at[pl.ds(base, rows)],
                                embb.at[pl.ds(0, rows)])
        else:
            pltpu.sync_copy(emb_hbm.at[pl.ds(base, rows)],
                            embb.at[pl.ds(0, rows)])

        def grp(g, carry):
            d16 = (plsc.load_gather(p0b, [g * 16 + iota16, zero16])
                   + plsc.load_gather(p1b, [g * 16 + iota16, zero16]))
            disb[pl.ds(g * 16, 16)] = _rsqrt16(d16)
            return carry

        lax.fori_loop(0, rows // 16, grp, 0)

        def rowgrp(g, carry):
            dv16 = disb[pl.ds(g * 16, 16)]
            for l in range(16):
                r = g * 16 + l
                dv = dv16[l]
                for q in range(D // 16):
                    embb[r, pl.ds(q * 16, 16)] = dv * embb[r, pl.ds(q * 16, 16)]
            return carry

        lax.fori_loop(0, rows // 16, rowgrp, 0)
        pltpu.sync_copy(embb.at[pl.ds(0, rows)], s0_out.at[pl.ds(base, rows)])
        pltpu.sync_copy(disb.at[pl.ds(0, rows)], dis_out.at[pl.ds(base, rows)])
        off += rows


def _make_scale(final):
    outs = (jax.ShapeDtypeStruct((N_NODES, D), jnp.float32)
            if final else
            (jax.ShapeDtypeStruct((NPAD, D), jnp.float32),
             jax.ShapeDtypeStruct((NPAD, D), jnp.float32)))

    @functools.partial(
        pl.kernel,
        out_type=outs,
        mesh=_MESH,
        scratch_types=[
            pltpu.VMEM((_CHUNKS[0], D), jnp.float32),
            pltpu.VMEM((_CHUNKS[0], D), jnp.float32),
            pltpu.VMEM((_CHUNKS[0],), jnp.float32),
        ],
        compiler_params=_SC_PARAMS_NL,
    )
    def _scale(acc_hbm, dis_hbm, sum_hbm, *refs):
        if final:
            out_hbm, accb, sumb, disb = refs
            s_out = sum_out = None
        else:
            s_out, sum_out, accb, sumb, disb = refs
        cid = lax.axis_index("c")
        sid = lax.axis_index("s")
        wid = cid * NTILE + sid
        w31 = wid == 31
        raw_sum = sum_hbm.shape[0] == N_NODES
        r0 = wid * STRIPE
        off = 0
        for rows in _CHUNKS:
            base = r0 + off
            pltpu.sync_copy(acc_hbm.at[pl.ds(base, rows)], accb.at[pl.ds(0, rows)])
            if raw_sum and off + rows == STRIPE:
                last = N_NODES - 31 * STRIPE - off

                @pl.when(w31)
                def _():
                    pltpu.sync_copy(sum_hbm.at[pl.ds(base, last)],
                                    sumb.at[pl.ds(0, last)])

                @pl.when(jnp.logical_not(w31))
                def _():
                    pltpu.sync_copy(sum_hbm.at[pl.ds(base, rows)],
                                    sumb.at[pl.ds(0, rows)])
            else:
                pltpu.sync_copy(sum_hbm.at[pl.ds(base, rows)],
                                sumb.at[pl.ds(0, rows)])
            pltpu.sync_copy(dis_hbm.at[pl.ds(base, rows)], disb.at[pl.ds(0, rows)])

            def rowgrp(g, carry):
                dv16 = disb[pl.ds(g * 16, 16)]
                for l in range(16):
                    r = g * 16 + l
                    dv = dv16[l]
                    for q in range(D // 16):
                        sl = pl.ds(q * 16, 16)
                        da = dv * accb[r, sl]
                        if final:
                            sumb[r, sl] = (sumb[r, sl] + da) * 0.25
                        else:
                            sumb[r, sl] = sumb[r, sl] + da
                            accb[r, sl] = dv * da
                return carry

            lax.fori_loop(0, rows // 16, rowgrp, 0)
            if final:
                if off + rows == STRIPE:
                    last = N_NODES - 31 * STRIPE - off

                    @pl.when(w31)
                    def _():
                        pltpu.sync_copy(sumb.at[pl.ds(0, last)],
                                        out_hbm.at[pl.ds(base, last)])

                    @pl.when(jnp.logical_not(w31))
                    def _():
                        pltpu.sync_copy(sumb.at[pl.ds(0, rows)],
                                        out_hbm.at[pl.ds(base, rows)])
                else:
                    pltpu.sync_copy(sumb.at[pl.ds(0, rows)],
                                    out_hbm.at[pl.ds(base, rows)])
            else:
                pltpu.sync_copy(accb.at[pl.ds(0, rows)],
                                s_out.at[pl.ds(base, rows)])
                pltpu.sync_copy(sumb.at[pl.ds(0, rows)],
                                sum_out.at[pl.ds(base, rows)])
            off += rows

    return _scale


_scale_sc = _make_scale(False)
_final_sc = _make_scale(True)


def kernel(edge_index, embedding):
    row = edge_index[0].astype(jnp.int32)
    col = edge_index[1].astype(jnp.int32)
    pad_e = EP - E
    col_p2 = jnp.concatenate(
        [col, jnp.full((pad_e,), NPAD, jnp.int32)]).reshape(ER, C)
    z64 = jnp.zeros((ACC, D), jnp.float32)
    z16 = jnp.zeros((ACCD, 16), jnp.float32)
    ones16 = jnp.ones((C, 16), jnp.float32)

    degt = _deg_sc(col_p2, z16, ones16)
    rowP, colP, cntP = _part_sc(row, col)
    s0, dis = _norm_sc(degt, embedding)
    acc1 = _layer_sc(s0, rowP, colP, cntP, z64)
    s1, summ = _scale_sc(acc1, dis, embedding)
    acc2 = _layer_sc(s1, rowP, colP, cntP, z64)
    s2, summ = _scale_sc(acc2, dis, summ)
    acc3 = _layer_sc(s2, rowP, colP, cntP, z64)
    return _final_sc(acc3, dis, summ)
